# baseline (speedup 1.0000x reference)
import functools

import jax
import jax.numpy as jnp
from jax import lax
from jax.experimental import pallas as pl
from jax.experimental.pallas import tpu as pltpu
from jax.experimental.pallas import tpu_sc as plsc

N_NODES = 10000
N_EDGES = 320000
D = 128

NC = 2
NS = 16
NW = NC * NS
L = 16

E_C = 128
N_CHUNKS = N_EDGES // E_C
N_FULL = N_CHUNKS // NW
N_LEFT = N_CHUNKS - N_FULL * NW
NB = 2
NI = 6
N_STEADY = 12
R_S = 624
R_REM = N_NODES - NS * R_S


def _sc_segment_sum(embed, edge_index, att):
    mesh = plsc.VectorSubcoreMesh(core_axis_name="c", subcore_axis_name="s")

    @functools.partial(
        pl.kernel,
        out_type=jax.ShapeDtypeStruct((NC, N_NODES, D), jnp.float32),
        mesh=mesh,
        scratch_types=[
            pltpu.VMEM((NI, 2, E_C), jnp.int32),
            pltpu.VMEM((NI, E_C), jnp.float32),
            pltpu.VMEM((NB, E_C, D), jnp.float32),
            pltpu.VMEM_SHARED((N_NODES, D), jnp.float32),
            [pltpu.SemaphoreType.DMA] * NI,
            [pltpu.SemaphoreType.DMA] * NB,
            [pltpu.SemaphoreType.DMA] * NB,
        ],
    )
    def k(embed_hbm, ei_hbm, att_hbm, out_hbm,
          ei_v, att_v, rows_v, acc, isems, gsems, ssems):
        cid = lax.axis_index("c")
        sid = lax.axis_index("s")
        wid = sid * NC + cid

        zero = jnp.zeros((L,), jnp.float32)

        def zero_row(r, _):
            for j in range(D // L):
                rows_v[0, r, pl.ds(j * L, L)] = zero
            return _

        lax.fori_loop(0, E_C, zero_row, None)
        row0 = sid * R_S
        for t in range(R_S // E_C):
            pltpu.sync_copy(rows_v.at[0], acc.at[pl.ds(row0 + t * E_C, E_C)])
        rem = R_S - (R_S // E_C) * E_C
        if rem:
            pltpu.sync_copy(rows_v.at[0, pl.ds(0, rem)],
                            acc.at[pl.ds(row0 + (R_S // E_C) * E_C, rem)])

        @pl.when(sid == NS - 1)
        def _():
            pltpu.sync_copy(rows_v.at[0, pl.ds(0, R_REM)],
                            acc.at[pl.ds(NS * R_S, R_REM)])

        plsc.subcore_barrier()

        def scale(b, i):
            def scale_block(kk, _):
                att16 = att_v[i, pl.ds(kk * L, L)]
                for l in range(L):
                    a = att16[l]
                    for j in range(D // L):
                        sl = pl.ds(j * L, L)
                        rows_v[b, kk * L + l, sl] = rows_v[b, kk * L + l, sl] * a
                return _

            lax.fori_loop(0, E_C // L, scale_block, None)

        def ebase(c):
            return (c * NW + wid) * E_C

        def stage_idx(c, i):
            base = ebase(c)
            pltpu.async_copy(ei_hbm.at[pl.ds(0, 2), pl.ds(base, E_C)],
                             ei_v.at[i], isems[i])
            pltpu.async_copy(att_hbm.at[pl.ds(base, E_C)], att_v.at[i],
                             isems[i])

        def wait_idx(c, i):
            base = ebase(c)
            pltpu.make_async_copy(ei_hbm.at[pl.ds(0, 2), pl.ds(base, E_C)],
                                  ei_v.at[i], isems[i]).wait()
            pltpu.make_async_copy(att_hbm.at[pl.ds(base, E_C)], att_v.at[i],
                                  isems[i]).wait()

        def gather(b, i):
            pltpu.async_copy(embed_hbm.at[ei_v.at[i, 0]], rows_v.at[b],
                             gsems[b])

        def wait_gather(b, i):
            pltpu.make_async_copy(embed_hbm.at[ei_v.at[i, 0]], rows_v.at[b],
                                  gsems[b]).wait()

        def scatter(b, i):
            pltpu.async_copy(rows_v.at[b], acc.at[ei_v.at[i, 1]], ssems[b],
                             add=True)

        def wait_scatter(b, i):
            pltpu.make_async_copy(rows_v.at[b], acc.at[ei_v.at[i, 1]],
                                  ssems[b]).wait()

        for c0 in range(3):
            stage_idx(c0, c0)
        wait_idx(0, 0)
        gather(0, 0)
        wait_gather(0, 0)
        scale(0, 0)
        scatter(0, 0)
        wait_idx(1, 1)
        wait_idx(2, 2)
        gather(1 % NB, 1)
        stage_idx(3, 3)

        def steady(t, _):
            c = 6 * t + 1
            for u in range(6):
                b = (1 + u) % NB
                i = (1 + u) % NI
                cc = c + u
                wait_gather(b, i)
                scale(b, i)
                scatter(b, i)
                wait_idx(cc + 2, (i + 2) % NI)
                wait_scatter((b + 1) % NB, (i - 1) % NI)
                gather((b + 1) % NB, (i + 1) % NI)
                stage_idx(cc + 3, (i + 3) % NI)
            return _

        lax.fori_loop(0, N_STEADY, steady, None)

        for c in range(73, 78):
            b = c % NB
            i = c % NI
            wait_gather(b, i)
            scale(b, i)
            scatter(b, i)
            if c + 2 <= 77:
                wait_idx(c + 2, (c + 2) % NI)
            if c + 1 <= 77:
                wait_scatter((c + 1) % NB, (c - 1) % NI)
                gather((c + 1) % NB, (c + 1) % NI)
            if c + 3 <= 77:
                stage_idx(c + 3, (c + 3) % NI)
        wait_scatter(0, 76 % NI)

        @pl.when(wid < N_LEFT)
        def _():
            base = (N_FULL * NW + wid) * E_C
            pltpu.sync_copy(ei_hbm.at[pl.ds(0, 2), pl.ds(base, E_C)],
                            ei_v.at[0])
            pltpu.async_copy(att_hbm.at[pl.ds(base, E_C)], att_v.at[0],
                             gsems[0])
            pltpu.async_copy(embed_hbm.at[ei_v.at[0, 0]], rows_v.at[0],
                             gsems[0])
            pltpu.make_async_copy(att_hbm.at[pl.ds(base, E_C)], att_v.at[0],
                                  gsems[0]).wait()
            pltpu.make_async_copy(embed_hbm.at[ei_v.at[0, 0]], rows_v.at[0],
                                  gsems[0]).wait()
            scale(0, 0)
            scatter(0, 0)
            wait_scatter(0, 0)

        wait_scatter(1, 77 % NI)

        plsc.subcore_barrier()
        pltpu.sync_copy(acc.at[pl.ds(row0, R_S)],
                        out_hbm.at[cid, pl.ds(row0, R_S)])

        @pl.when(sid == NS - 1)
        def _():
            pltpu.sync_copy(acc.at[pl.ds(NS * R_S, R_REM)],
                            out_hbm.at[cid, pl.ds(NS * R_S, R_REM)])

    return k(embed, edge_index, att)


def _tc_tail_body(e_ref, p_ref, w_ref, b_ref, o_ref):
    h = e_ref[...] + p_ref[0] + p_ref[1]
    y = lax.dot_general(h, w_ref[...], (((1,), (1,)), ((), ())),
                        preferred_element_type=jnp.float32)
    y = y + b_ref[...]
    o_ref[...] = jnp.where(y >= 0, y, 0.01 * y)


def _tc_tail(embed, partials, W_w, W_b):
    BR = 1000
    grid = N_NODES // BR
    return pl.pallas_call(
        _tc_tail_body,
        grid=(grid,),
        in_specs=[
            pl.BlockSpec((BR, D), lambda i: (i, 0)),
            pl.BlockSpec((NC, BR, D), lambda i: (0, i, 0)),
            pl.BlockSpec((D, D), lambda i: (0, 0)),
            pl.BlockSpec((1, D), lambda i: (0, 0)),
        ],
        out_specs=pl.BlockSpec((BR, D), lambda i: (i, 0)),
        out_shape=jax.ShapeDtypeStruct((N_NODES, D), jnp.float32),
    )(embed, partials, W_w, W_b)


@jax.jit
def kernel(entity_embed, edge_index, edge_att, W_w, W_b):
    ei = edge_index.astype(jnp.int32)
    partials = _sc_segment_sum(entity_embed, ei, edge_att)
    return _tc_tail(entity_embed, partials, W_w, W_b.reshape(1, D))


# speedup vs baseline: 1.2525x; 1.2525x over previous
import functools

import jax
import jax.numpy as jnp
from jax import lax
from jax.experimental import pallas as pl
from jax.experimental.pallas import tpu as pltpu
from jax.experimental.pallas import tpu_sc as plsc

N_NODES = 10000
N_EDGES = 320000
D = 128

NC = 2
NS = 16
NW = NC * NS
L = 16

E_C = 128
N_CHUNKS = N_EDGES // E_C
N_FULL = N_CHUNKS // NW
N_LEFT = N_CHUNKS - N_FULL * NW
NB = 2
NI = 6
N_STEADY = 12
R_S = 624
R_REM = N_NODES - NS * R_S


def _sc_segment_sum(embed, edge_index, att):
    mesh = plsc.VectorSubcoreMesh(core_axis_name="c", subcore_axis_name="s")

    @functools.partial(
        pl.kernel,
        out_type=jax.ShapeDtypeStruct((NC, N_NODES, D), jnp.float32),
        mesh=mesh,
        scratch_types=[
            pltpu.VMEM((NI, 2, E_C), jnp.int32),
            pltpu.VMEM((NI, E_C), jnp.float32),
            pltpu.VMEM((NB, E_C, D), jnp.float32),
            pltpu.VMEM_SHARED((N_NODES, D), jnp.float32),
            [pltpu.SemaphoreType.DMA] * NI,
            [pltpu.SemaphoreType.DMA] * NB,
            [pltpu.SemaphoreType.DMA] * NB,
        ],
    )
    def k(embed_hbm, ei_hbm, att_hbm, out_hbm,
          ei_v, att_v, rows_v, acc, isems, gsems, ssems):
        cid = lax.axis_index("c")
        sid = lax.axis_index("s")
        wid = sid * NC + cid

        zero = jnp.zeros((L,), jnp.float32)

        def zero_row(r, _):
            for j in range(D // L):
                rows_v[0, r, pl.ds(j * L, L)] = zero
            return _

        lax.fori_loop(0, E_C, zero_row, None)
        row0 = sid * R_S
        for t in range(R_S // E_C):
            pltpu.sync_copy(rows_v.at[0], acc.at[pl.ds(row0 + t * E_C, E_C)])
        rem = R_S - (R_S // E_C) * E_C
        if rem:
            pltpu.sync_copy(rows_v.at[0, pl.ds(0, rem)],
                            acc.at[pl.ds(row0 + (R_S // E_C) * E_C, rem)])

        @pl.when(sid == NS - 1)
        def _():
            pltpu.sync_copy(rows_v.at[0, pl.ds(0, R_REM)],
                            acc.at[pl.ds(NS * R_S, R_REM)])

        plsc.subcore_barrier()

        def scale(b, i):
            def scale_block(kk, _):
                att16 = att_v[i, pl.ds(kk * L, L)]
                for l in range(L):
                    a = att16[l]
                    for j in range(D // L):
                        sl = pl.ds(j * L, L)
                        rows_v[b, kk * L + l, sl] = rows_v[b, kk * L + l, sl] * a
                return _

            lax.fori_loop(0, E_C // L, scale_block, None)

        def ebase(c):
            return (c * NW + wid) * E_C

        def stage_idx(c, i):
            base = ebase(c)
            pltpu.async_copy(ei_hbm.at[pl.ds(0, 2), pl.ds(base, E_C)],
                             ei_v.at[i], isems[i])
            pltpu.async_copy(att_hbm.at[pl.ds(base, E_C)], att_v.at[i],
                             isems[i])

        def wait_idx(c, i):
            base = ebase(c)
            pltpu.make_async_copy(ei_hbm.at[pl.ds(0, 2), pl.ds(base, E_C)],
                                  ei_v.at[i], isems[i]).wait()
            pltpu.make_async_copy(att_hbm.at[pl.ds(base, E_C)], att_v.at[i],
                                  isems[i]).wait()

        def gather(b, i):
            pltpu.async_copy(embed_hbm.at[ei_v.at[i, 0]], rows_v.at[b],
                             gsems[b])

        def wait_gather(b, i):
            pltpu.make_async_copy(embed_hbm.at[ei_v.at[i, 0]], rows_v.at[b],
                                  gsems[b]).wait()

        def scatter(b, i):
            pltpu.async_copy(rows_v.at[b], acc.at[ei_v.at[i, 1]], ssems[b],
                             add=True)

        def wait_scatter(b, i):
            pltpu.make_async_copy(rows_v.at[b], acc.at[ei_v.at[i, 1]],
                                  ssems[b]).wait()

        for c0 in range(3):
            stage_idx(c0, c0)
        wait_idx(0, 0)
        gather(0, 0)
        wait_gather(0, 0)
        wait_idx(1, 1)
        gather(1 % NB, 1)
        scale(0, 0)
        scatter(0, 0)
        wait_idx(2, 2)
        stage_idx(3, 3)

        def steady(t, _):
            c = 6 * t + 1
            for u in range(6):
                b = (1 + u) % NB
                i = (1 + u) % NI
                cc = c + u
                wait_gather(b, i)
                wait_scatter((b + 1) % NB, (i - 1) % NI)
                gather((b + 1) % NB, (i + 1) % NI)
                scale(b, i)
                scatter(b, i)
                wait_idx(cc + 2, (i + 2) % NI)
                stage_idx(cc + 3, (i + 3) % NI)
            return _

        lax.fori_loop(0, N_STEADY, steady, None)

        for c in range(73, 78):
            b = c % NB
            i = c % NI
            wait_gather(b, i)
            wait_scatter((c + 1) % NB, (c - 1) % NI)
            if c + 1 <= 77:
                gather((c + 1) % NB, (c + 1) % NI)
            scale(b, i)
            scatter(b, i)
            if c + 2 <= 77:
                wait_idx(c + 2, (c + 2) % NI)
            if c + 3 <= 77:
                stage_idx(c + 3, (c + 3) % NI)

        @pl.when(wid < N_LEFT)
        def _():
            base = (N_FULL * NW + wid) * E_C
            pltpu.sync_copy(ei_hbm.at[pl.ds(0, 2), pl.ds(base, E_C)],
                            ei_v.at[0])
            pltpu.async_copy(att_hbm.at[pl.ds(base, E_C)], att_v.at[0],
                             gsems[0])
            pltpu.async_copy(embed_hbm.at[ei_v.at[0, 0]], rows_v.at[0],
                             gsems[0])
            pltpu.make_async_copy(att_hbm.at[pl.ds(base, E_C)], att_v.at[0],
                                  gsems[0]).wait()
            pltpu.make_async_copy(embed_hbm.at[ei_v.at[0, 0]], rows_v.at[0],
                                  gsems[0]).wait()
            scale(0, 0)
            scatter(0, 0)
            wait_scatter(0, 0)

        wait_scatter(1, 77 % NI)

        plsc.subcore_barrier()
        pltpu.sync_copy(acc.at[pl.ds(row0, R_S)],
                        out_hbm.at[cid, pl.ds(row0, R_S)])

        @pl.when(sid == NS - 1)
        def _():
            pltpu.sync_copy(acc.at[pl.ds(NS * R_S, R_REM)],
                            out_hbm.at[cid, pl.ds(NS * R_S, R_REM)])

    return k(embed, edge_index, att)


def _tc_tail_body(e_ref, p_ref, w_ref, b_ref, o_ref):
    h = e_ref[...] + p_ref[0] + p_ref[1]
    y = lax.dot_general(h, w_ref[...], (((1,), (1,)), ((), ())),
                        preferred_element_type=jnp.float32)
    y = y + b_ref[...]
    o_ref[...] = jnp.where(y >= 0, y, 0.01 * y)


def _tc_tail(embed, partials, W_w, W_b):
    BR = 1000
    grid = N_NODES // BR
    return pl.pallas_call(
        _tc_tail_body,
        grid=(grid,),
        in_specs=[
            pl.BlockSpec((BR, D), lambda i: (i, 0)),
            pl.BlockSpec((NC, BR, D), lambda i: (0, i, 0)),
            pl.BlockSpec((D, D), lambda i: (0, 0)),
            pl.BlockSpec((1, D), lambda i: (0, 0)),
        ],
        out_specs=pl.BlockSpec((BR, D), lambda i: (i, 0)),
        out_shape=jax.ShapeDtypeStruct((N_NODES, D), jnp.float32),
    )(embed, partials, W_w, W_b)


@jax.jit
def kernel(entity_embed, edge_index, edge_att, W_w, W_b):
    ei = edge_index.astype(jnp.int32)
    partials = _sc_segment_sum(entity_embed, ei, edge_att)
    return _tc_tail(entity_embed, partials, W_w, W_b.reshape(1, D))


# speedup vs baseline: 1.2951x; 1.0340x over previous
import functools

import jax
import jax.numpy as jnp
from jax import lax
from jax.experimental import pallas as pl
from jax.experimental.pallas import tpu as pltpu
from jax.experimental.pallas import tpu_sc as plsc

N_NODES = 10000
N_EDGES = 320000
D = 128

NC = 2
NS = 16
NW = NC * NS
L = 16

E_C = 128
N_CHUNKS = N_EDGES // E_C
N_FULL = N_CHUNKS // NW
E_W = N_FULL * E_C
N_LEFT = N_CHUNKS - N_FULL * NW
NB = 2
N_STEADY = (N_FULL - 2) // NB
R_S = 624
R_REM = N_NODES - NS * R_S


def _sc_segment_sum(embed, edge_index, att):
    mesh = plsc.VectorSubcoreMesh(core_axis_name="c", subcore_axis_name="s")

    @functools.partial(
        pl.kernel,
        out_type=jax.ShapeDtypeStruct((NC, N_NODES, D), jnp.float32),
        mesh=mesh,
        scratch_types=[
            pltpu.VMEM((1, E_W), jnp.int32),
            pltpu.VMEM((NB, 1, E_C), jnp.int32),
            pltpu.VMEM((NB, E_C), jnp.float32),
            pltpu.VMEM((NB, E_C, D), jnp.float32),
            pltpu.VMEM_SHARED((N_NODES, D), jnp.float32),
            [pltpu.SemaphoreType.DMA] * NB,
            [pltpu.SemaphoreType.DMA] * NB,
        ],
    )
    def k(embed_hbm, ei_hbm, att_hbm, out_hbm,
          src_v, dst_v, att_v, rows_v, acc, gsems, ssems):
        cid = lax.axis_index("c")
        sid = lax.axis_index("s")
        wid = sid * NC + cid

        zero = jnp.zeros((L,), jnp.float32)

        def zero_row(r, _):
            for j in range(D // L):
                rows_v[0, r, pl.ds(j * L, L)] = zero
            return _

        lax.fori_loop(0, E_C, zero_row, None)
        row0 = sid * R_S
        for t in range(R_S // E_C):
            pltpu.sync_copy(rows_v.at[0], acc.at[pl.ds(row0 + t * E_C, E_C)])
        rem = R_S - (R_S // E_C) * E_C
        if rem:
            pltpu.sync_copy(rows_v.at[0, pl.ds(0, rem)],
                            acc.at[pl.ds(row0 + (R_S // E_C) * E_C, rem)])

        @pl.when(sid == NS - 1)
        def _():
            pltpu.sync_copy(rows_v.at[0, pl.ds(0, R_REM)],
                            acc.at[pl.ds(NS * R_S, R_REM)])

        pltpu.sync_copy(ei_hbm.at[pl.ds(0, 1), pl.ds(wid * E_W, E_W)], src_v)
        plsc.subcore_barrier()

        def scale(b):
            def scale_block(kk, _):
                att16 = att_v[b, pl.ds(kk * L, L)]
                for l in range(L):
                    a = att16[l]
                    for j in range(D // L):
                        sl = pl.ds(j * L, L)
                        rows_v[b, kk * L + l, sl] = rows_v[b, kk * L + l, sl] * a
                return _

            lax.fori_loop(0, E_C // L, scale_block, None)

        def gather(c, b):
            base = wid * E_W + c * E_C
            pltpu.async_copy(ei_hbm.at[pl.ds(1, 1), pl.ds(base, E_C)],
                             dst_v.at[b], gsems[b])
            pltpu.async_copy(att_hbm.at[pl.ds(base, E_C)], att_v.at[b], gsems[b])
            pltpu.async_copy(embed_hbm.at[src_v.at[0, pl.ds(c * E_C, E_C)]],
                             rows_v.at[b], gsems[b])

        def wait_gather(c, b):
            base = wid * E_W + c * E_C
            pltpu.make_async_copy(ei_hbm.at[pl.ds(1, 1), pl.ds(base, E_C)],
                                  dst_v.at[b], gsems[b]).wait()
            pltpu.make_async_copy(att_hbm.at[pl.ds(base, E_C)], att_v.at[b],
                                  gsems[b]).wait()
            pltpu.make_async_copy(embed_hbm.at[src_v.at[0, pl.ds(c * E_C, E_C)]],
                                  rows_v.at[b], gsems[b]).wait()

        def scatter(b):
            pltpu.async_copy(rows_v.at[b], acc.at[dst_v.at[b, 0]], ssems[b],
                             add=True)

        def wait_scatter(b):
            pltpu.make_async_copy(rows_v.at[b], acc.at[dst_v.at[b, 0]],
                                  ssems[b]).wait()

        gather(0, 0)
        wait_gather(0, 0)
        gather(1, 1)
        scale(0)
        scatter(0)

        def steady(t, _):
            c = NB * t + 1
            for u in range(NB):
                b = (1 + u) % NB
                cc = c + u
                wait_gather(cc, b)
                wait_scatter((b + 1) % NB)
                gather(cc + 1, (b + 1) % NB)
                scale(b)
                scatter(b)
            return _

        lax.fori_loop(0, N_STEADY, steady, None)

        wait_gather(N_FULL - 1, (N_FULL - 1) % NB)
        wait_scatter(N_FULL % NB)
        scale((N_FULL - 1) % NB)
        scatter((N_FULL - 1) % NB)

        @pl.when(wid < N_LEFT)
        def _():
            base = (N_FULL * NW + wid) * E_C
            pltpu.sync_copy(ei_hbm.at[pl.ds(0, 1), pl.ds(base, E_C)],
                            src_v.at[pl.ds(0, 1), pl.ds(0, E_C)])
            pltpu.async_copy(ei_hbm.at[pl.ds(1, 1), pl.ds(base, E_C)],
                             dst_v.at[0], gsems[0])
            pltpu.async_copy(att_hbm.at[pl.ds(base, E_C)], att_v.at[0],
                             gsems[0])
            pltpu.async_copy(embed_hbm.at[src_v.at[0, pl.ds(0, E_C)]],
                             rows_v.at[0], gsems[0])
            pltpu.make_async_copy(ei_hbm.at[pl.ds(1, 1), pl.ds(base, E_C)],
                                  dst_v.at[0], gsems[0]).wait()
            pltpu.make_async_copy(att_hbm.at[pl.ds(base, E_C)], att_v.at[0],
                                  gsems[0]).wait()
            pltpu.make_async_copy(embed_hbm.at[src_v.at[0, pl.ds(0, E_C)]],
                                  rows_v.at[0], gsems[0]).wait()
            scale(0)
            scatter(0)
            wait_scatter(0)

        wait_scatter((N_FULL - 1) % NB)

        plsc.subcore_barrier()
        pltpu.sync_copy(acc.at[pl.ds(row0, R_S)],
                        out_hbm.at[cid, pl.ds(row0, R_S)])

        @pl.when(sid == NS - 1)
        def _():
            pltpu.sync_copy(acc.at[pl.ds(NS * R_S, R_REM)],
                            out_hbm.at[cid, pl.ds(NS * R_S, R_REM)])

    return k(embed, edge_index, att)


def _tc_tail_body(e_ref, p_ref, w_ref, b_ref, o_ref):
    h = e_ref[...] + p_ref[0] + p_ref[1]
    y = lax.dot_general(h, w_ref[...], (((1,), (1,)), ((), ())),
                        preferred_element_type=jnp.float32)
    y = y + b_ref[...]
    o_ref[...] = jnp.where(y >= 0, y, 0.01 * y)


def _tc_tail(embed, partials, W_w, W_b):
    BR = 1000
    grid = N_NODES // BR
    return pl.pallas_call(
        _tc_tail_body,
        grid=(grid,),
        in_specs=[
            pl.BlockSpec((BR, D), lambda i: (i, 0)),
            pl.BlockSpec((NC, BR, D), lambda i: (0, i, 0)),
            pl.BlockSpec((D, D), lambda i: (0, 0)),
            pl.BlockSpec((1, D), lambda i: (0, 0)),
        ],
        out_specs=pl.BlockSpec((BR, D), lambda i: (i, 0)),
        out_shape=jax.ShapeDtypeStruct((N_NODES, D), jnp.float32),
    )(embed, partials, W_w, W_b)


@jax.jit
def kernel(entity_embed, edge_index, edge_att, W_w, W_b):
    ei = edge_index.astype(jnp.int32)
    partials = _sc_segment_sum(entity_embed, ei, edge_att)
    return _tc_tail(entity_embed, partials, W_w, W_b.reshape(1, D))


# speedup vs baseline: 1.3131x; 1.0139x over previous
import functools

import jax
import jax.numpy as jnp
from jax import lax
from jax.experimental import pallas as pl
from jax.experimental.pallas import tpu as pltpu
from jax.experimental.pallas import tpu_sc as plsc

N_NODES = 10000
N_EDGES = 320000
D = 128

NC = 2
NS = 16
NW = NC * NS
L = 16

E_W = N_EDGES // NW
E_C = 80
N_CHUNKS = E_W // E_C
NB = 3
N_STEADY = (N_CHUNKS - 5) // NB
R_S = 624
R_REM = N_NODES - NS * R_S


def _sc_segment_sum(embed, src1, dst1, att1):
    mesh = plsc.VectorSubcoreMesh(core_axis_name="c", subcore_axis_name="s")

    @functools.partial(
        pl.kernel,
        out_type=jax.ShapeDtypeStruct((NC, N_NODES, D), jnp.float32),
        mesh=mesh,
        scratch_types=[
            pltpu.VMEM((E_W,), jnp.int32),
            pltpu.VMEM((NB, E_C), jnp.int32),
            pltpu.VMEM((NB, E_C), jnp.float32),
            pltpu.VMEM((NB, E_C, D), jnp.float32),
            pltpu.VMEM_SHARED((N_NODES, D), jnp.float32),
            [pltpu.SemaphoreType.DMA] * NB,
            [pltpu.SemaphoreType.DMA] * NB,
        ],
    )
    def k(embed_hbm, src_hbm, dst_hbm, att_hbm, out_hbm,
          src_v, dst_v, att_v, rows_v, acc, gsems, ssems):
        cid = lax.axis_index("c")
        sid = lax.axis_index("s")
        wid = sid * NC + cid

        zero = jnp.zeros((L,), jnp.float32)

        def zero_row(r, _):
            for j in range(D // L):
                rows_v[0, r, pl.ds(j * L, L)] = zero
            return _

        lax.fori_loop(0, E_C, zero_row, None)
        row0 = sid * R_S
        for t in range(R_S // E_C):
            pltpu.sync_copy(rows_v.at[0], acc.at[pl.ds(row0 + t * E_C, E_C)])
        rem = R_S - (R_S // E_C) * E_C
        if rem:
            pltpu.sync_copy(rows_v.at[0, pl.ds(0, rem)],
                            acc.at[pl.ds(row0 + (R_S // E_C) * E_C, rem)])

        @pl.when(sid == NS - 1)
        def _():
            pltpu.sync_copy(rows_v.at[0, pl.ds(0, R_REM)],
                            acc.at[pl.ds(NS * R_S, R_REM)])

        pltpu.sync_copy(src_hbm.at[pl.ds(wid * E_W, E_W)], src_v)
        plsc.subcore_barrier()

        def scale(b):
            def scale_block(kk, _):
                att16 = att_v[b, pl.ds(kk * L, L)]
                for l in range(L):
                    a = att16[l]
                    for j in range(D // L):
                        sl = pl.ds(j * L, L)
                        rows_v[b, kk * L + l, sl] = rows_v[b, kk * L + l, sl] * a
                return _

            lax.fori_loop(0, E_C // L, scale_block, None)

        def gather(c, b):
            base = wid * E_W + c * E_C
            pltpu.async_copy(dst_hbm.at[pl.ds(base, E_C)], dst_v.at[b], gsems[b])
            pltpu.async_copy(att_hbm.at[pl.ds(base, E_C)], att_v.at[b], gsems[b])
            pltpu.async_copy(embed_hbm.at[src_v.at[pl.ds(c * E_C, E_C)]],
                             rows_v.at[b], gsems[b])

        def wait_gather(c, b):
            base = wid * E_W + c * E_C
            pltpu.make_async_copy(dst_hbm.at[pl.ds(base, E_C)], dst_v.at[b],
                                  gsems[b]).wait()
            pltpu.make_async_copy(att_hbm.at[pl.ds(base, E_C)], att_v.at[b],
                                  gsems[b]).wait()
            pltpu.make_async_copy(embed_hbm.at[src_v.at[pl.ds(c * E_C, E_C)]],
                                  rows_v.at[b], gsems[b]).wait()

        def scatter(b):
            pltpu.async_copy(rows_v.at[b], acc.at[dst_v.at[b]], ssems[b],
                             add=True)

        def wait_scatter(b):
            pltpu.make_async_copy(rows_v.at[b], acc.at[dst_v.at[b]],
                                  ssems[b]).wait()

        def proc(c, b, nxt, wait_prev_scatter, issue_next):
            wait_gather(c, b)
            scale(b)
            scatter(b)
            if issue_next:
                if wait_prev_scatter:
                    wait_scatter(nxt)
                gather(c + 2, nxt)

        gather(0, 0)
        gather(1, 1)
        proc(0, 0, 2, False, True)
        proc(1, 1, 0, True, True)
        proc(2, 2, 1, True, True)

        def steady(t, _):
            c = NB * t + NB
            for u in range(NB):
                b = u
                nxt = (u + 2) % NB
                cc = c + u
                wait_gather(cc, b)
                scale(b)
                scatter(b)
                wait_scatter(nxt)
                gather(cc + 2, nxt)
            return _

        lax.fori_loop(0, N_STEADY, steady, None)

        proc(N_CHUNKS - 2, (N_CHUNKS - 2) % NB, 0, False, False)
        proc(N_CHUNKS - 1, (N_CHUNKS - 1) % NB, 0, False, False)
        for b in range(NB):
            wait_scatter(b)

        plsc.subcore_barrier()
        pltpu.sync_copy(acc.at[pl.ds(row0, R_S)],
                        out_hbm.at[cid, pl.ds(row0, R_S)])

        @pl.when(sid == NS - 1)
        def _():
            pltpu.sync_copy(acc.at[pl.ds(NS * R_S, R_REM)],
                            out_hbm.at[cid, pl.ds(NS * R_S, R_REM)])

    return k(embed, src1, dst1, att1)


def _tc_tail_body(e_ref, p_ref, w_ref, b_ref, o_ref):
    h = e_ref[...] + p_ref[0] + p_ref[1]
    y = lax.dot_general(h, w_ref[...], (((1,), (1,)), ((), ())),
                        preferred_element_type=jnp.float32)
    y = y + b_ref[...]
    o_ref[...] = jnp.where(y >= 0, y, 0.01 * y)


def _tc_tail(embed, partials, W_w, W_b):
    BR = 1000
    grid = N_NODES // BR
    return pl.pallas_call(
        _tc_tail_body,
        grid=(grid,),
        in_specs=[
            pl.BlockSpec((BR, D), lambda i: (i, 0)),
            pl.BlockSpec((NC, BR, D), lambda i: (0, i, 0)),
            pl.BlockSpec((D, D), lambda i: (0, 0)),
            pl.BlockSpec((1, D), lambda i: (0, 0)),
        ],
        out_specs=pl.BlockSpec((BR, D), lambda i: (i, 0)),
        out_shape=jax.ShapeDtypeStruct((N_NODES, D), jnp.float32),
    )(embed, partials, W_w, W_b)


@jax.jit
def kernel(entity_embed, edge_index, edge_att, W_w, W_b):
    src = edge_index[0].astype(jnp.int32)
    dst = edge_index[1].astype(jnp.int32)
    att = edge_att
    partials = _sc_segment_sum(entity_embed, src, dst, att)
    return _tc_tail(entity_embed, partials, W_w, W_b.reshape(1, D))
